# Initial kernel scaffold; baseline (speedup 1.0000x reference)
import functools

import jax
import jax.numpy as jnp
from jax import lax
from jax.experimental import pallas as pl
from jax.experimental.pallas import tpu as pltpu
from jax.experimental.pallas import tpu_sc as plsc

N = 10000
E = 320000
D = 128
G = 64
NC = 2
NS = 16
NW = NC * NS
E_PER_W = E // NW
CH = 80
NCHUNK = E_PER_W // CH
RPT = N // NS

BLK = 2000
NB = N // BLK

_mesh = plsc.VectorSubcoreMesh(core_axis_name="c", subcore_axis_name="s")


@functools.partial(
    pl.kernel,
    out_type=jax.ShapeDtypeStruct((NC, N, 16), jnp.float32),
    mesh=_mesh,
    scratch_types=[
        pltpu.VMEM((CH,), jnp.int32),
        pltpu.VMEM((CH, 16), jnp.float32),
        pltpu.VMEM_SHARED((N, 16), jnp.float32),
    ],
)
def _deg_pass(dst_hbm, z16_hbm, ones_hbm, out_hbm, idx_v, ones_v, acc_sh):
    c = lax.axis_index("c")
    s = lax.axis_index("s")
    rbase = s * RPT
    pltpu.sync_copy(z16_hbm.at[pl.ds(rbase, RPT)], acc_sh.at[pl.ds(rbase, RPT)])
    pltpu.sync_copy(ones_hbm, ones_v)
    plsc.subcore_barrier()
    ebase = (c * NS + s) * E_PER_W

    def body(i, carry):
        pltpu.sync_copy(dst_hbm.at[pl.ds(ebase + i * CH, CH)], idx_v)
        pltpu.sync_copy(ones_v, acc_sh.at[idx_v], add=True)
        return carry

    lax.fori_loop(0, NCHUNK, body, 0)
    plsc.subcore_barrier()
    pltpu.sync_copy(acc_sh.at[pl.ds(rbase, RPT)], out_hbm.at[c, pl.ds(rbase, RPT)])


@functools.partial(
    pl.kernel,
    out_type=jax.ShapeDtypeStruct((NC, N, D), jnp.float32),
    mesh=_mesh,
    scratch_types=[
        pltpu.VMEM((CH,), jnp.int32),
        pltpu.VMEM((CH,), jnp.int32),
        pltpu.VMEM((CH, D), jnp.float32),
        pltpu.VMEM_SHARED((N, D), jnp.float32),
        pltpu.SemaphoreType.DMA,
    ],
)
def _edge_pass(g_hbm, src_hbm, dst_hbm, zD_hbm, out_hbm,
               src_v, dst_v, rows_v, acc_sh, sem):
    c = lax.axis_index("c")
    s = lax.axis_index("s")
    rbase = s * RPT
    pltpu.sync_copy(zD_hbm.at[pl.ds(rbase, RPT)], acc_sh.at[pl.ds(rbase, RPT)])
    plsc.subcore_barrier()
    ebase = (c * NS + s) * E_PER_W

    def body(i, carry):
        off = ebase + i * CH
        pltpu.sync_copy(src_hbm.at[pl.ds(off, CH)], src_v)
        pltpu.sync_copy(dst_hbm.at[pl.ds(off, CH)], dst_v)
        pltpu.async_copy(g_hbm.at[src_v], rows_v, sem).wait()
        pltpu.sync_copy(rows_v, acc_sh.at[dst_v], add=True)
        return carry

    lax.fori_loop(0, NCHUNK, body, 0)
    plsc.subcore_barrier()
    pltpu.sync_copy(acc_sh.at[pl.ds(rbase, RPT)], out_hbm.at[c, pl.ds(rbase, RPT)])


def _dis_from(degp_ref):
    deg = degp_ref[0, :, 0:1] + degp_ref[1, :, 0:1] + 1.0
    return lax.rsqrt(deg)


def _g1_body(x_ref, w_ref, degp_ref, o_ref):
    dis = _dis_from(degp_ref)
    h = lax.dot_general(x_ref[...], w_ref[...], (((1,), (1,)), ((), ())),
                        preferred_element_type=jnp.float32,
                        precision=lax.Precision.HIGHEST)
    o_ref[...] = h * dis


def _g2_body(p_ref, g1_ref, degp_ref, w_ref, o_ref):
    dis = _dis_from(degp_ref)
    tot = (p_ref[0] + p_ref[1] + g1_ref[...]) * dis
    u = jnp.where(tot >= 0, tot, 0.01 * tot)
    h = lax.dot_general(u, w_ref[...], (((1,), (1,)), ((), ())),
                        preferred_element_type=jnp.float32,
                        precision=lax.Precision.HIGHEST)
    o_ref[...] = h * dis


def _pool_body(p_ref, g2_ref, degp_ref, b_ref, o_ref):
    i = pl.program_id(0)
    dis = _dis_from(degp_ref)
    h2 = (p_ref[0] + p_ref[1] + g2_ref[...]) * dis
    b = b_ref[0, 0, :]
    gids = lax.broadcasted_iota(jnp.int32, (G, BLK), 0)
    sel = (b[None, :] == gids).astype(jnp.float32)
    contrib = lax.dot_general(sel, h2, (((1,), (0,)), ((), ())),
                              preferred_element_type=jnp.float32,
                              precision=lax.Precision.HIGHEST)

    @pl.when(i == 0)
    def _():
        o_ref[...] = contrib

    @pl.when(i > 0)
    def _():
        o_ref[...] += contrib


_g1_call = pl.pallas_call(
    _g1_body,
    grid=(NB,),
    in_specs=[
        pl.BlockSpec((BLK, D), lambda i: (i, 0)),
        pl.BlockSpec((D, D), lambda i: (0, 0)),
        pl.BlockSpec((NC, BLK, 16), lambda i: (0, i, 0)),
    ],
    out_specs=pl.BlockSpec((BLK, D), lambda i: (i, 0)),
    out_shape=jax.ShapeDtypeStruct((N, D), jnp.float32),
)

_g2_call = pl.pallas_call(
    _g2_body,
    grid=(NB,),
    in_specs=[
        pl.BlockSpec((NC, BLK, D), lambda i: (0, i, 0)),
        pl.BlockSpec((BLK, D), lambda i: (i, 0)),
        pl.BlockSpec((NC, BLK, 16), lambda i: (0, i, 0)),
        pl.BlockSpec((D, D), lambda i: (0, 0)),
    ],
    out_specs=pl.BlockSpec((BLK, D), lambda i: (i, 0)),
    out_shape=jax.ShapeDtypeStruct((N, D), jnp.float32),
)

_pool_call = pl.pallas_call(
    _pool_body,
    grid=(NB,),
    in_specs=[
        pl.BlockSpec((NC, BLK, D), lambda i: (0, i, 0)),
        pl.BlockSpec((BLK, D), lambda i: (i, 0)),
        pl.BlockSpec((NC, BLK, 16), lambda i: (0, i, 0)),
        pl.BlockSpec((1, 1, BLK), lambda i: (i, 0, 0)),
    ],
    out_specs=pl.BlockSpec((G, D), lambda i: (0, 0)),
    out_shape=jax.ShapeDtypeStruct((G, D), jnp.float32),
)


def kernel(x, edge_index, batch, W1, W2):
    src = edge_index[0]
    dst = edge_index[1]
    z16 = jnp.zeros((N, 16), jnp.float32)
    ones16 = jnp.ones((CH, 16), jnp.float32)
    zD = jnp.zeros((N, D), jnp.float32)
    batch3 = batch.reshape(NB, 1, BLK)

    degp = _deg_pass(dst, z16, ones16)
    g1 = _g1_call(x, W1, degp)
    p1 = _edge_pass(g1, src, dst, zD)
    g2 = _g2_call(p1, g1, degp, W2)
    p2 = _edge_pass(g2, src, dst, zD)
    return _pool_call(p2, g2, degp, batch3)


# speedup vs baseline: 12.3628x; 12.3628x over previous
import functools

import jax
import jax.numpy as jnp
from jax import lax
from jax.experimental import pallas as pl
from jax.experimental.pallas import tpu as pltpu
from jax.experimental.pallas import tpu_sc as plsc

N = 10000
E = 320000
D = 128
G = 64
NC = 2
NS = 16
NW = NC * NS
E_PER_W = E // NW
CH = 80
NCHUNK = E_PER_W // CH
RPT = 624
RTAIL = N - NS * RPT
RTOFF = NS * RPT

BLK = 2000
NB = N // BLK

_mesh = plsc.VectorSubcoreMesh(core_axis_name="c", subcore_axis_name="s")


@functools.partial(
    pl.kernel,
    out_type=jax.ShapeDtypeStruct((NC, N, D), jnp.float32),
    mesh=_mesh,
    scratch_types=[
        pltpu.VMEM((CH,), jnp.int32),
        pltpu.VMEM((CH, D), jnp.float32),
        pltpu.VMEM_SHARED((N, D), jnp.float32),
    ],
)
def _deg_pass(dst_hbm, zD_hbm, ones_hbm, out_hbm, idx_v, ones_v, acc_sh):
    c = lax.axis_index("c")
    s = lax.axis_index("s")
    rbase = s * RPT
    pltpu.sync_copy(zD_hbm.at[pl.ds(rbase, RPT)], acc_sh.at[pl.ds(rbase, RPT)])

    @pl.when(s == NS - 1)
    def _():
        pltpu.sync_copy(zD_hbm.at[pl.ds(RTOFF, RTAIL)],
                        acc_sh.at[pl.ds(RTOFF, RTAIL)])

    pltpu.sync_copy(ones_hbm, ones_v)
    plsc.subcore_barrier()
    ebase = (c * NS + s) * E_PER_W

    def body(i, carry):
        pltpu.sync_copy(dst_hbm.at[pl.ds(ebase + i * CH, CH)], idx_v)
        pltpu.sync_copy(ones_v, acc_sh.at[idx_v], add=True)
        return carry

    lax.fori_loop(0, NCHUNK, body, 0)
    plsc.subcore_barrier()
    pltpu.sync_copy(acc_sh.at[pl.ds(rbase, RPT)], out_hbm.at[c, pl.ds(rbase, RPT)])

    @pl.when(s == NS - 1)
    def _():
        pltpu.sync_copy(acc_sh.at[pl.ds(RTOFF, RTAIL)],
                        out_hbm.at[c, pl.ds(RTOFF, RTAIL)])


@functools.partial(
    pl.kernel,
    out_type=jax.ShapeDtypeStruct((NC, N, D), jnp.float32),
    mesh=_mesh,
    scratch_types=[
        pltpu.VMEM((CH,), jnp.int32),
        pltpu.VMEM((CH,), jnp.int32),
        pltpu.VMEM((CH, D), jnp.float32),
        pltpu.VMEM_SHARED((N, D), jnp.float32),
        pltpu.SemaphoreType.DMA,
    ],
)
def _edge_pass(g_hbm, src_hbm, dst_hbm, zD_hbm, out_hbm,
               src_v, dst_v, rows_v, acc_sh, sem):
    c = lax.axis_index("c")
    s = lax.axis_index("s")
    rbase = s * RPT
    pltpu.sync_copy(zD_hbm.at[pl.ds(rbase, RPT)], acc_sh.at[pl.ds(rbase, RPT)])

    @pl.when(s == NS - 1)
    def _():
        pltpu.sync_copy(zD_hbm.at[pl.ds(RTOFF, RTAIL)],
                        acc_sh.at[pl.ds(RTOFF, RTAIL)])

    plsc.subcore_barrier()
    ebase = (c * NS + s) * E_PER_W

    def body(i, carry):
        off = ebase + i * CH
        pltpu.sync_copy(src_hbm.at[pl.ds(off, CH)], src_v)
        pltpu.sync_copy(dst_hbm.at[pl.ds(off, CH)], dst_v)
        pltpu.async_copy(g_hbm.at[src_v], rows_v, sem).wait()
        pltpu.sync_copy(rows_v, acc_sh.at[dst_v], add=True)
        return carry

    lax.fori_loop(0, NCHUNK, body, 0)
    plsc.subcore_barrier()
    pltpu.sync_copy(acc_sh.at[pl.ds(rbase, RPT)], out_hbm.at[c, pl.ds(rbase, RPT)])

    @pl.when(s == NS - 1)
    def _():
        pltpu.sync_copy(acc_sh.at[pl.ds(RTOFF, RTAIL)],
                        out_hbm.at[c, pl.ds(RTOFF, RTAIL)])


def _dis_from(degp_ref):
    deg = degp_ref[0, :, 0:1] + degp_ref[1, :, 0:1] + 1.0
    return lax.rsqrt(deg)


def _g1_body(x_ref, w_ref, degp_ref, o_ref):
    dis = _dis_from(degp_ref)
    h = lax.dot_general(x_ref[...], w_ref[...], (((1,), (1,)), ((), ())),
                        preferred_element_type=jnp.float32,
                        precision=lax.Precision.HIGHEST)
    o_ref[...] = h * dis


def _g2_body(p_ref, g1_ref, degp_ref, w_ref, o_ref):
    dis = _dis_from(degp_ref)
    tot = (p_ref[0] + p_ref[1] + g1_ref[...]) * dis
    u = jnp.where(tot >= 0, tot, 0.01 * tot)
    h = lax.dot_general(u, w_ref[...], (((1,), (1,)), ((), ())),
                        preferred_element_type=jnp.float32,
                        precision=lax.Precision.HIGHEST)
    o_ref[...] = h * dis


def _pool_body(p_ref, g2_ref, degp_ref, b_ref, o_ref):
    i = pl.program_id(0)
    dis = _dis_from(degp_ref)
    h2 = (p_ref[0] + p_ref[1] + g2_ref[...]) * dis
    b = b_ref[0, 0, :]
    gids = lax.broadcasted_iota(jnp.int32, (G, BLK), 0)
    sel = (b[None, :] == gids).astype(jnp.float32)
    contrib = lax.dot_general(sel, h2, (((1,), (0,)), ((), ())),
                              preferred_element_type=jnp.float32,
                              precision=lax.Precision.HIGHEST)

    @pl.when(i == 0)
    def _():
        o_ref[...] = contrib

    @pl.when(i > 0)
    def _():
        o_ref[...] += contrib


_g1_call = pl.pallas_call(
    _g1_body,
    grid=(NB,),
    in_specs=[
        pl.BlockSpec((BLK, D), lambda i: (i, 0)),
        pl.BlockSpec((D, D), lambda i: (0, 0)),
        pl.BlockSpec((NC, BLK, 16), lambda i: (0, i, 0)),
    ],
    out_specs=pl.BlockSpec((BLK, D), lambda i: (i, 0)),
    out_shape=jax.ShapeDtypeStruct((N, D), jnp.float32),
)

_g2_call = pl.pallas_call(
    _g2_body,
    grid=(NB,),
    in_specs=[
        pl.BlockSpec((NC, BLK, D), lambda i: (0, i, 0)),
        pl.BlockSpec((BLK, D), lambda i: (i, 0)),
        pl.BlockSpec((NC, BLK, 16), lambda i: (0, i, 0)),
        pl.BlockSpec((D, D), lambda i: (0, 0)),
    ],
    out_specs=pl.BlockSpec((BLK, D), lambda i: (i, 0)),
    out_shape=jax.ShapeDtypeStruct((N, D), jnp.float32),
)

_pool_call = pl.pallas_call(
    _pool_body,
    grid=(NB,),
    in_specs=[
        pl.BlockSpec((NC, BLK, D), lambda i: (0, i, 0)),
        pl.BlockSpec((BLK, D), lambda i: (i, 0)),
        pl.BlockSpec((NC, BLK, 16), lambda i: (0, i, 0)),
        pl.BlockSpec((1, 1, BLK), lambda i: (i, 0, 0)),
    ],
    out_specs=pl.BlockSpec((G, D), lambda i: (0, 0)),
    out_shape=jax.ShapeDtypeStruct((G, D), jnp.float32),
)


def kernel(x, edge_index, batch, W1, W2):
    src = edge_index[0]
    dst = edge_index[1]
    onesD = jnp.ones((CH, D), jnp.float32)
    zD = jnp.zeros((N, D), jnp.float32)
    batch3 = batch.reshape(NB, 1, BLK)

    degp_w = _deg_pass(dst, zD, onesD)
    degp = lax.slice(degp_w, (0, 0, 0), (NC, N, 16))
    g1 = _g1_call(x, W1, degp)
    p1 = _edge_pass(g1, src, dst, zD)
    g2 = _g2_call(p1, g1, degp, W2)
    p2 = _edge_pass(g2, src, dst, zD)
    return _pool_call(p2, g2, degp, batch3)
